# baseline (speedup 1.0000x reference)
import functools

import jax
import jax.numpy as jnp
from jax import lax
from jax.experimental import pallas as pl
from jax.experimental.pallas import tpu as pltpu
from jax.experimental.pallas import tpu_sc as plsc

BATCH = 16384
D = 64
NREL = 1000
NW = 32
BPW = BATCH // NW
CH = 128
NCHUNK = BPW // CH


def _sc_gather_prod(heads_hbm, tails_hbm, ent_h_hbm, ent_t_hbm, out_hbm,
                    idx_h, idx_t, hh, tt, ht, th, prod, sem):
    wid = lax.axis_index("s") * 2 + lax.axis_index("c")
    base = wid * BPW
    pltpu.sync_copy(heads_hbm.at[pl.ds(base, BPW)], idx_h)
    pltpu.sync_copy(tails_hbm.at[pl.ds(base, BPW)], idx_t)
    for ci in range(NCHUNK):
        off = ci * CH
        ih = idx_h.at[pl.ds(off, CH)]
        it = idx_t.at[pl.ds(off, CH)]
        cp1 = pltpu.async_copy(ent_h_hbm.at[ih], hh, sem)
        cp2 = pltpu.async_copy(ent_t_hbm.at[it], tt, sem)
        cp3 = pltpu.async_copy(ent_h_hbm.at[it], ht, sem)
        cp4 = pltpu.async_copy(ent_t_hbm.at[ih], th, sem)
        cp1.wait()
        cp2.wait()
        cp3.wait()
        cp4.wait()

        def row_body(r, _):
            for j in range(D // 16):
                s = pl.ds(16 * j, 16)
                prod[r, pl.ds(16 * j, 16)] = hh[r, s] * tt[r, s]
                prod[r, pl.ds(D + 16 * j, 16)] = ht[r, s] * th[r, s]
            return 0

        lax.fori_loop(0, CH, row_body, 0)
        pltpu.sync_copy(prod, out_hbm.at[pl.ds(base + off, CH)])


def _tc_score(x_ref, w_ref, o_ref):
    acc = jnp.dot(x_ref[...], w_ref[...], preferred_element_type=jnp.float32)
    o_ref[...] = jnp.clip(acc * 0.5, -20.0, 20.0)


def kernel(pairs, ent_h, ent_t, rel, rel_inv):
    heads = pairs[:, 0].astype(jnp.int32)
    tails = pairs[:, 1].astype(jnp.int32)

    mesh = plsc.VectorSubcoreMesh(core_axis_name="c", subcore_axis_name="s")
    sc_fn = functools.partial(
        pl.kernel,
        mesh=mesh,
        out_type=jax.ShapeDtypeStruct((BATCH, 2 * D), jnp.float32),
        scratch_types=[
            pltpu.VMEM((BPW,), jnp.int32),
            pltpu.VMEM((BPW,), jnp.int32),
            pltpu.VMEM((CH, D), jnp.float32),
            pltpu.VMEM((CH, D), jnp.float32),
            pltpu.VMEM((CH, D), jnp.float32),
            pltpu.VMEM((CH, D), jnp.float32),
            pltpu.VMEM((CH, 2 * D), jnp.float32),
            pltpu.SemaphoreType.DMA,
        ],
        compiler_params=pltpu.CompilerParams(use_tc_tiling_on_sc=False),
    )(_sc_gather_prod)
    prod = sc_fn(heads, tails, ent_h, ent_t)

    w = jnp.concatenate([rel, rel_inv], axis=1).T

    bb = 512
    out = pl.pallas_call(
        _tc_score,
        grid=(BATCH // bb,),
        in_specs=[
            pl.BlockSpec((bb, 2 * D), lambda i: (i, 0)),
            pl.BlockSpec((2 * D, NREL), lambda i: (0, 0)),
        ],
        out_specs=pl.BlockSpec((bb, NREL), lambda i: (i, 0)),
        out_shape=jax.ShapeDtypeStruct((BATCH, NREL), jnp.float32),
    )(prod, w)
    return out


# speedup vs baseline: 1.4663x; 1.4663x over previous
import functools

import jax
import jax.numpy as jnp
from jax import lax
from jax.experimental import pallas as pl
from jax.experimental.pallas import tpu as pltpu
from jax.experimental.pallas import tpu_sc as plsc

BATCH = 16384
D = 64
NREL = 1000
NW = 32
BPW = BATCH // NW
CH = 16
NCHUNK = BPW // CH


def _sc_gather_prod(h_hbm, t_hbm, ent_h, ent_t, out_hbm,
                    idx_h, idx_t, hh, tt, ht, th, prod, sem):
    wid = lax.axis_index("s") * 2 + lax.axis_index("c")
    base = wid * BPW
    pltpu.sync_copy(h_hbm.at[pl.ds(base, BPW)], idx_h)
    pltpu.sync_copy(t_hbm.at[pl.ds(base, BPW)], idx_t)

    def chunk(ci, _):
        off = pl.multiple_of(ci * CH, CH)
        hv = idx_h[pl.ds(off, CH)]
        tv = idx_t[pl.ds(off, CH)]
        cps = []
        for r in range(CH):
            hs = hv[r]
            ts = tv[r]
            cps.append(pltpu.async_copy(ent_h.at[hs], hh.at[r], sem))
            cps.append(pltpu.async_copy(ent_t.at[ts], tt.at[r], sem))
            cps.append(pltpu.async_copy(ent_h.at[ts], ht.at[r], sem))
            cps.append(pltpu.async_copy(ent_t.at[hs], th.at[r], sem))
        for cp in cps:
            cp.wait()
        for r in range(CH):
            for j in range(D // 16):
                s = pl.ds(16 * j, 16)
                prod[r, pl.ds(16 * j, 16)] = hh[r, s] * tt[r, s]
                prod[r, pl.ds(D + 16 * j, 16)] = ht[r, s] * th[r, s]
        pltpu.sync_copy(prod, out_hbm.at[pl.ds(base + off, CH)])
        return 0

    lax.fori_loop(0, NCHUNK, chunk, 0)


def _tc_score(x_ref, w_ref, o_ref):
    acc = jnp.dot(x_ref[...], w_ref[...], preferred_element_type=jnp.float32)
    o_ref[...] = jnp.clip(acc * 0.5, -20.0, 20.0)


def kernel(pairs, ent_h, ent_t, rel, rel_inv):
    heads = pairs[:, 0].astype(jnp.int32)
    tails = pairs[:, 1].astype(jnp.int32)

    mesh = plsc.VectorSubcoreMesh(core_axis_name="c", subcore_axis_name="s")
    sc_fn = functools.partial(
        pl.kernel,
        mesh=mesh,
        out_type=jax.ShapeDtypeStruct((BATCH, 2 * D), jnp.float32),
        scratch_types=[
            pltpu.VMEM((BPW,), jnp.int32),
            pltpu.VMEM((BPW,), jnp.int32),
            pltpu.VMEM((CH, D), jnp.float32),
            pltpu.VMEM((CH, D), jnp.float32),
            pltpu.VMEM((CH, D), jnp.float32),
            pltpu.VMEM((CH, D), jnp.float32),
            pltpu.VMEM((CH, 2 * D), jnp.float32),
            pltpu.SemaphoreType.DMA,
        ],
        compiler_params=pltpu.CompilerParams(use_tc_tiling_on_sc=True),
    )(_sc_gather_prod)
    prod = sc_fn(heads, tails, ent_h, ent_t)

    w = jnp.concatenate([rel, rel_inv], axis=1).T

    bb = 512
    out = pl.pallas_call(
        _tc_score,
        grid=(BATCH // bb,),
        in_specs=[
            pl.BlockSpec((bb, 2 * D), lambda i: (i, 0)),
            pl.BlockSpec((2 * D, NREL), lambda i: (0, 0)),
        ],
        out_specs=pl.BlockSpec((bb, NREL), lambda i: (i, 0)),
        out_shape=jax.ShapeDtypeStruct((BATCH, NREL), jnp.float32),
    )(prod, w)
    return out
